# Initial kernel scaffold; baseline (speedup 1.0000x reference)
import jax
import jax.numpy as jnp
from jax.experimental import pallas as pl


def kernel(centers, centers_features, cls_preds):
    raise NotImplementedError("write your pallas kernel here")


# speedup vs baseline: 137.3243x; 137.3243x over previous
import jax
import jax.numpy as jnp
from jax.experimental import pallas as pl
from jax.experimental.pallas import tpu as pltpu

_CLASS_RADIUS = (0.8, 0.6, 0.4)
_N = 5000
_NP = 5120
_B = 512
_NBLK = _NP // _B


def _nms_body(dataR_ref, dataT_ref, rank_ref, combo_ref,
              keep_out_ref, combo_out_ref, keep_s):
    keep_s[...] = jnp.ones((1, _NP), jnp.float32)
    dataR = dataR_ref[...][0]
    dataT = dataT_ref[...][0]

    for blk in range(_NBLK):
        i0 = blk * _B
        w = _NP - i0
        dx = dataR[i0:i0 + _B, 0:1] - dataT[0:1, i0:]
        dy = dataR[i0:i0 + _B, 1:2] - dataT[1:2, i0:]
        dz = dataR[i0:i0 + _B, 2:3] - dataT[2:3, i0:]
        d2 = dx * dx + dy * dy + dz * dz
        ri = dataR[i0:i0 + _B, 3:4]
        li = dataR[i0:i0 + _B, 4:5]
        lj = dataT[4:5, i0:]
        ii = jax.lax.broadcasted_iota(jnp.int32, (_B, w), 0)
        jj = jax.lax.broadcasted_iota(jnp.int32, (_B, w), 1)
        sup = (d2 < ri * ri) & (li == lj) & (jj > ii)
        supf = sup.astype(jnp.float32)

        s_intra = supf[:, :_B]
        k0 = keep_s[0:1, i0:i0 + _B]
        k0_8 = jnp.broadcast_to(k0, (8, _B))

        def cond(carry):
            _, done = carry
            return jnp.logical_not(done)

        def body(carry):
            k, _ = carry
            t = jnp.dot(k, s_intra, preferred_element_type=jnp.float32)
            k_new = k0_8 * (t == 0.0).astype(jnp.float32)
            return k_new, jnp.all(k_new == k)

        k_fix, _ = jax.lax.while_loop(cond, body, (k0_8, jnp.bool_(False)))

        counts = jnp.dot(k_fix, supf, preferred_element_type=jnp.float32)
        alive = (counts[0:1, :] == 0.0).astype(jnp.float32)
        keep_s[0:1, i0:] = keep_s[0:1, i0:] * alive

    keep_row = keep_s[0:1, :]
    for blk in range(_NBLK):
        i0 = blk * _B
        rk = rank_ref[0, i0:i0 + _B, 0:1]
        jidx = jax.lax.broadcasted_iota(jnp.int32, (_B, _NP), 1)
        onehot = (rk == jidx).astype(jnp.float32)
        kb = jnp.sum(onehot * keep_row, axis=1, keepdims=True)
        keep_out_ref[0, i0:i0 + _B, :] = kb
        combo_out_ref[0, i0:i0 + _B, :] = combo_ref[0, i0:i0 + _B, :] * kb


def _mask_body(f_ref, k_ref, o_ref):
    o_ref[...] = f_ref[...] * k_ref[...]


def kernel(centers, centers_features, cls_preds):
    bt, n, _ = centers.shape
    radii = jnp.array(_CLASS_RADIUS, dtype=jnp.float32)

    scores = jnp.max(cls_preds, axis=-1)
    labels = jnp.argmax(cls_preds, axis=-1).astype(jnp.int32)
    order = jnp.argsort(-scores, axis=-1)
    rank = jnp.argsort(order, axis=-1).astype(jnp.int32)

    c_s = jnp.take_along_axis(centers, order[..., None], axis=1)
    l_s = jnp.take_along_axis(labels, order, axis=1)
    r_s = radii[l_s]

    pad = _NP - n
    xyz = jnp.pad(c_s, ((0, 0), (0, pad), (0, 0)), constant_values=1e9)
    rr = jnp.pad(r_s, ((0, 0), (0, pad)))[..., None]
    ll = jnp.pad(l_s.astype(jnp.float32), ((0, 0), (0, pad)),
                 constant_values=-1.0)[..., None]
    z3 = jnp.zeros((bt, _NP, 3), jnp.float32)
    dataR = jnp.concatenate([xyz, rr, ll, z3], axis=-1)
    dataT = jnp.transpose(dataR, (0, 2, 1))

    rankp = jnp.pad(rank, ((0, 0), (0, pad)), constant_values=n)[..., None]

    combo = jnp.concatenate(
        [centers, cls_preds, jnp.zeros((bt, n, 2), jnp.float32)], axis=-1)
    combo = jnp.pad(combo, ((0, 0), (0, pad), (0, 0)))

    keep_col, combo_out = pl.pallas_call(
        _nms_body,
        grid=(bt,),
        in_specs=[
            pl.BlockSpec((1, _NP, 8), lambda b: (b, 0, 0)),
            pl.BlockSpec((1, 8, _NP), lambda b: (b, 0, 0)),
            pl.BlockSpec((1, _NP, 1), lambda b: (b, 0, 0)),
            pl.BlockSpec((1, _NP, 8), lambda b: (b, 0, 0)),
        ],
        out_specs=[
            pl.BlockSpec((1, _NP, 1), lambda b: (b, 0, 0)),
            pl.BlockSpec((1, _NP, 8), lambda b: (b, 0, 0)),
        ],
        out_shape=[
            jax.ShapeDtypeStruct((bt, _NP, 1), jnp.float32),
            jax.ShapeDtypeStruct((bt, _NP, 8), jnp.float32),
        ],
        scratch_shapes=[pltpu.VMEM((1, _NP), jnp.float32)],
    )(dataR, dataT, rankp, combo)

    keep_n = keep_col[:, :n, :]
    filtered_centers = combo_out[:, :n, 0:3]
    filtered_cls = combo_out[:, :n, 3:6]

    rblk = 1000
    feat_out = pl.pallas_call(
        _mask_body,
        grid=(bt, n // rblk),
        in_specs=[
            pl.BlockSpec((1, rblk, 512), lambda b, j: (b, j, 0)),
            pl.BlockSpec((1, rblk, 1), lambda b, j: (b, j, 0)),
        ],
        out_specs=pl.BlockSpec((1, rblk, 512), lambda b, j: (b, j, 0)),
        out_shape=jax.ShapeDtypeStruct((bt, n, 512), jnp.float32),
    )(centers_features, keep_n)

    keep = keep_n[..., 0] > 0.5
    return filtered_centers, feat_out, filtered_cls, keep
